# baseline (speedup 1.0000x reference)
import jax
import jax.numpy as jnp
from jax.experimental import pallas as pl
from jax.experimental.pallas import tpu as pltpu

_MODALITY = 3
_BLOCK_ROWS = 2048


def _add_row_kernel(v_ref, t_ref, o_ref):
    o_ref[...] = v_ref[...] + t_ref[0]


def kernel(val, table, key_ids):
    B, L, D = val.shape
    rows = B * L
    v2 = val.reshape(rows, D)
    blk = _BLOCK_ROWS
    grid = (rows // blk,)
    out = pl.pallas_call(
        _add_row_kernel,
        grid=grid,
        in_specs=[
            pl.BlockSpec((blk, D), lambda i: (i, 0)),
            pl.BlockSpec((1, 1, D), lambda i: (_MODALITY, 0, 0)),
        ],
        out_specs=pl.BlockSpec((blk, D), lambda i: (i, 0)),
        out_shape=jax.ShapeDtypeStruct((rows, D), val.dtype),
        compiler_params=pltpu.CompilerParams(
            dimension_semantics=("arbitrary",),
        ),
    )(v2, table.reshape(8, 1, D))
    return out.reshape(B, L, D)


# speedup vs baseline: 1.0181x; 1.0181x over previous
import jax
import jax.numpy as jnp
from jax.experimental import pallas as pl
from jax.experimental.pallas import tpu as pltpu

_MODALITY = 3
_BLOCK_ROWS = 2048


def _add_row_kernel(v_ref, t_ref, o_ref):
    o_ref[...] = v_ref[...] + t_ref[_MODALITY:_MODALITY + 1, :]


def kernel(val, table, key_ids):
    B, L, D = val.shape
    rows = B * L
    v2 = val.reshape(rows, D)
    blk = _BLOCK_ROWS
    grid = (rows // blk,)
    out = pl.pallas_call(
        _add_row_kernel,
        grid=grid,
        in_specs=[
            pl.BlockSpec((blk, D), lambda i: (i, 0)),
            pl.BlockSpec((8, D), lambda i: (0, 0)),
        ],
        out_specs=pl.BlockSpec((blk, D), lambda i: (i, 0)),
        out_shape=jax.ShapeDtypeStruct((rows, D), val.dtype),
        compiler_params=pltpu.CompilerParams(
            dimension_semantics=("parallel",),
        ),
    )(v2, table)
    return out.reshape(B, L, D)
